# Initial kernel scaffold; baseline (speedup 1.0000x reference)
import jax
import jax.numpy as jnp
from jax.experimental import pallas as pl


def kernel(g_emb, g_candidates_emb, batch_idx, W0, b0, W1, b1, W2, b2, Wf, bf):
    raise NotImplementedError("write your pallas kernel here")


# speedup vs baseline: 4.5807x; 4.5807x over previous
import functools

import jax
import jax.numpy as jnp
import numpy as np
from jax.experimental import pallas as pl
from jax.experimental.pallas import tpu as pltpu

B = 128
N = 65536
EMB = 64
HID = 128
EPS = 1e-4

BLK = 2048
NBLK = N // BLK

_NEG_INF = np.float32(-np.inf)
_TINY = np.float32(np.finfo(np.float32).tiny)


def _u32(x):
    return jnp.uint32(x)


def _threefry_bits(m):
    k1 = _u32(0)
    k2 = _u32(123)
    ks2 = k1 ^ k2 ^ _u32(0x1BD11BDA)

    def rotl(x, r):
        return (x << _u32(r)) | (x >> _u32(32 - r))

    x0 = jnp.zeros_like(m) + k1
    x1 = m + k2
    rots = ((13, 15, 26, 6), (17, 29, 16, 24))
    ks = (k1, k2, ks2)
    for i in range(5):
        for r in rots[i % 2]:
            x0 = x0 + x1
            x1 = rotl(x1, r)
            x1 = x1 ^ x0
        x0 = x0 + ks[(i + 1) % 3]
        x1 = x1 + ks[(i + 2) % 3] + _u32(i + 1)
    return x0 ^ x1


def _gumbel_from_bits(bits):
    fb = (bits >> _u32(9)) | _u32(0x3F800000)
    u01 = jax.lax.bitcast_convert_type(fb, jnp.float32) - jnp.float32(1.0)
    one_minus_tiny = jnp.float32(1.0) - _TINY
    u = jnp.maximum(_TINY, u01 * one_minus_tiny + _TINY)
    return -jnp.log(-jnp.log(u))


def _pass_a(g_emb_ref, bidx_ref, cand_ref, w0a_ref, w0b_ref, b0_ref,
            w1_ref, b1_ref, w2_ref, b2_ref, wf_ref, bf_ref,
            x_out_ref, logits_ref, gmax_ref, counts_ref):
    step = pl.program_id(0)
    bidx = bidx_ref[...]
    cand = cand_ref[...]
    gids = jax.lax.broadcasted_iota(jnp.int32, (1, B), 1)
    onehot = (bidx == gids).astype(jnp.float32)
    x_rep = jnp.dot(onehot, g_emb_ref[...],
                    preferred_element_type=jnp.float32)
    x_out_ref[...] = jnp.concatenate([x_rep, cand], axis=1)
    h = jnp.dot(x_rep, w0a_ref[...], preferred_element_type=jnp.float32)
    h = h + jnp.dot(cand, w0b_ref[...], preferred_element_type=jnp.float32)
    h = jnp.maximum(h + b0_ref[...], 0.0)
    h = jnp.maximum(jnp.dot(h, w1_ref[...], preferred_element_type=jnp.float32)
                    + b1_ref[...], 0.0)
    h = jnp.maximum(jnp.dot(h, w2_ref[...], preferred_element_type=jnp.float32)
                    + b2_ref[...], 0.0)
    logits = jnp.dot(h, wf_ref[...], preferred_element_type=jnp.float32) \
        + bf_ref[...]
    logits_ref[...] = logits

    masked = jnp.where(onehot > 0.0, logits, _NEG_INF)
    blk_max = jnp.max(masked, axis=0, keepdims=True)
    blk_cnt = jnp.sum(onehot, axis=0, keepdims=True)

    @pl.when(step == 0)
    def _init():
        gmax_ref[...] = jnp.full((1, B), _NEG_INF, jnp.float32)
        counts_ref[...] = jnp.zeros((1, B), jnp.float32)

    gmax_ref[...] = jnp.maximum(gmax_ref[...], blk_max)
    counts_ref[...] = counts_ref[...] + blk_cnt


def _pass_b(bidx_ref, logits_ref, gmax_ref, gsum_ref, ex_ref):
    step = pl.program_id(0)
    bidx = bidx_ref[...]
    gids = jax.lax.broadcasted_iota(jnp.int32, (1, B), 1)
    onehot = (bidx == gids).astype(jnp.float32)
    gmax_elem = jnp.sum(onehot * gmax_ref[...], axis=1,
                        keepdims=True)
    ex = jnp.exp(logits_ref[...] - gmax_elem)
    ex_ref[...] = ex
    blk_sum = jnp.sum(onehot * ex, axis=0, keepdims=True)

    @pl.when(step == 0)
    def _init():
        gsum_ref[...] = jnp.zeros((1, B), jnp.float32)

    gsum_ref[...] = gsum_ref[...] + blk_sum


def _pass_c(bidx_ref, logits_ref, ex_ref, gmax_ref, gsum_ref, counts_ref,
            probs_ref, lp_out_ref, act_out_ref, shift_out_ref,
            bval_ref, bidx_best_ref, bprob_ref, p0_ref):
    step = pl.program_id(0)
    bidx = bidx_ref[...]
    gids = jax.lax.broadcasted_iota(jnp.int32, (1, B), 1)
    onehot_b = (bidx == gids)
    onehot = onehot_b.astype(jnp.float32)
    gsum_elem = jnp.sum(onehot * gsum_ref[...], axis=1,
                        keepdims=True) + EPS
    probs = ex_ref[...] / gsum_elem
    probs_ref[...] = probs

    j_local = jax.lax.broadcasted_iota(jnp.int32, (BLK, 1), 0)
    j_global = j_local + step * BLK
    row = (B - 1) - bidx
    m = row * N + j_global
    bits = _threefry_bits(m.astype(jnp.uint32))
    gumbel = _gumbel_from_bits(bits)

    ok = probs > EPS
    val = jnp.where(ok, logits_ref[...] + gumbel, _NEG_INF)
    vmask = jnp.where(onehot_b, val, _NEG_INF)
    blk_max = jnp.max(vmask, axis=0, keepdims=True)
    is_max = (vmask == blk_max) & onehot_b & (blk_max > _NEG_INF)
    big = jnp.int32(N)
    jcol = jnp.where(is_max, j_local, big)
    blk_arg_local = jnp.min(jcol, axis=0, keepdims=True)
    first = (j_local == blk_arg_local) & is_max
    blk_prob = jnp.sum(jnp.where(first, probs, 0.0), axis=0, keepdims=True)
    blk_arg = jnp.where(blk_arg_local < big,
                        blk_arg_local + step * BLK, jnp.int32(0))

    @pl.when(step == 0)
    def _init():
        bval_ref[...] = jnp.full((1, B), _NEG_INF, jnp.float32)
        bidx_best_ref[...] = jnp.zeros((1, B), jnp.int32)
        bprob_ref[...] = jnp.zeros((1, B), jnp.float32)
        p0_ref[...] = jnp.full((1, B), probs[0, 0], jnp.float32)

    upd = blk_max > bval_ref[...]
    bval_ref[...] = jnp.where(upd, blk_max, bval_ref[...])
    bidx_best_ref[...] = jnp.where(upd, blk_arg, bidx_best_ref[...])
    bprob_ref[...] = jnp.where(upd, blk_prob, bprob_ref[...])

    @pl.when(step == NBLK - 1)
    def _fin():
        never = bval_ref[...] == _NEG_INF
        sel_prob = jnp.where(never, p0_ref[...], bprob_ref[...])
        sel_idx = jnp.where(never, 0, bidx_best_ref[...])
        rev = (jax.lax.broadcasted_iota(jnp.int32, (B, B), 0)
               + jax.lax.broadcasted_iota(jnp.int32, (B, B), 1)) == (B - 1)
        revf = rev.astype(jnp.float32)
        shifted_f = jnp.dot(sel_idx.astype(jnp.float32), revf,
                            preferred_element_type=jnp.float32)
        lp = jnp.log(jnp.dot(sel_prob, revf,
                             preferred_element_type=jnp.float32))
        tri = (jax.lax.broadcasted_iota(jnp.int32, (B, B), 0)
               < jax.lax.broadcasted_iota(jnp.int32, (B, B), 1))
        shift_f = jnp.dot(counts_ref[...], tri.astype(jnp.float32),
                          preferred_element_type=jnp.float32)
        shift_out_ref[...] = shifted_f.astype(jnp.int32)
        act_out_ref[...] = (shifted_f - shift_f).astype(jnp.int32)
        lp_out_ref[...] = lp


def kernel(g_emb, g_candidates_emb, batch_idx, W0, b0, W1, b1, W2, b2, Wf, bf):
    bidx2 = batch_idx.reshape(N, 1)
    w0a = W0[:EMB]
    w0b = W0[EMB:]
    b0r = b0.reshape(1, HID)
    b1r = b1.reshape(1, HID)
    b2r = b2.reshape(1, HID)
    bfr = bf.reshape(1, 1)

    row_spec = pl.BlockSpec((BLK, 1), lambda i: (i, 0))
    full = lambda shape: pl.BlockSpec(shape, lambda i: tuple(0 for _ in shape))
    acc = pl.BlockSpec((1, B), lambda i: (0, 0))

    x_states, logits, gmax, counts = pl.pallas_call(
        _pass_a,
        grid=(NBLK,),
        in_specs=[
            full((B, EMB)), row_spec,
            pl.BlockSpec((BLK, EMB), lambda i: (i, 0)),
            full((EMB, HID)), full((EMB, HID)), full((1, HID)),
            full((HID, HID)), full((1, HID)),
            full((HID, HID)), full((1, HID)),
            full((HID, 1)), full((1, 1)),
        ],
        out_specs=[
            pl.BlockSpec((BLK, 2 * EMB), lambda i: (i, 0)),
            row_spec, acc, acc,
        ],
        out_shape=[
            jax.ShapeDtypeStruct((N, 2 * EMB), jnp.float32),
            jax.ShapeDtypeStruct((N, 1), jnp.float32),
            jax.ShapeDtypeStruct((1, B), jnp.float32),
            jax.ShapeDtypeStruct((1, B), jnp.float32),
        ],
    )(g_emb, bidx2, g_candidates_emb, w0a, w0b, b0r, W1, b1r, W2, b2r, Wf, bfr)

    gsum, ex = pl.pallas_call(
        _pass_b,
        grid=(NBLK,),
        in_specs=[row_spec, row_spec, acc],
        out_specs=[acc, row_spec],
        out_shape=[
            jax.ShapeDtypeStruct((1, B), jnp.float32),
            jax.ShapeDtypeStruct((N, 1), jnp.float32),
        ],
    )(bidx2, logits, gmax)

    probs, logprobs, actions, shifted = pl.pallas_call(
        _pass_c,
        grid=(NBLK,),
        in_specs=[row_spec, row_spec, row_spec, acc, acc, acc],
        out_specs=[row_spec, acc, acc, acc],
        out_shape=[
            jax.ShapeDtypeStruct((N, 1), jnp.float32),
            jax.ShapeDtypeStruct((1, B), jnp.float32),
            jax.ShapeDtypeStruct((1, B), jnp.int32),
            jax.ShapeDtypeStruct((1, B), jnp.int32),
        ],
        scratch_shapes=[
            pltpu.VMEM((1, B), jnp.float32),
            pltpu.VMEM((1, B), jnp.int32),
            pltpu.VMEM((1, B), jnp.float32),
            pltpu.VMEM((1, B), jnp.float32),
        ],
    )(bidx2, logits, ex, gmax, gsum, counts)

    return (g_emb, x_states, probs.reshape(N),
            logprobs.reshape(B), actions.reshape(B), shifted.reshape(B))


# speedup vs baseline: 7.2092x; 1.5738x over previous
import functools

import jax
import jax.numpy as jnp
import numpy as np
from jax.experimental import pallas as pl
from jax.experimental.pallas import tpu as pltpu

B = 128
N = 65536
EMB = 64
HID = 128
EPS = 1e-4

BLK = 2048
NBLK = N // BLK

_NEG_INF = np.float32(-np.inf)
_TINY = np.float32(np.finfo(np.float32).tiny)


def _u32(x):
    return jnp.uint32(x)


def _threefry_bits(m):
    k1 = _u32(0)
    k2 = _u32(123)
    ks2 = k1 ^ k2 ^ _u32(0x1BD11BDA)

    def rotl(x, r):
        return (x << _u32(r)) | (x >> _u32(32 - r))

    x0 = jnp.zeros_like(m) + k1
    x1 = m + k2
    rots = ((13, 15, 26, 6), (17, 29, 16, 24))
    ks = (k1, k2, ks2)
    for i in range(5):
        for r in rots[i % 2]:
            x0 = x0 + x1
            x1 = rotl(x1, r)
            x1 = x1 ^ x0
        x0 = x0 + ks[(i + 1) % 3]
        x1 = x1 + ks[(i + 2) % 3] + _u32(i + 1)
    return x0 ^ x1


def _gumbel_from_bits(bits):
    fb = (bits >> _u32(9)) | _u32(0x3F800000)
    u01 = jax.lax.bitcast_convert_type(fb, jnp.float32) - jnp.float32(1.0)
    one_minus_tiny = jnp.float32(1.0) - _TINY
    u = jnp.maximum(_TINY, u01 * one_minus_tiny + _TINY)
    return -jnp.log(-jnp.log(u))


def _pass_a(g_emb_ref, bidx_ref, cand_ref, w0a_ref, w0b_ref, b0_ref,
            w1_ref, b1_ref, w2_ref, b2_ref, wf_ref, bf_ref,
            x_out_ref, logits_ref, gmax_ref, counts_ref):
    step = pl.program_id(0)
    bidx = bidx_ref[...]
    cand = cand_ref[...]
    gids = jax.lax.broadcasted_iota(jnp.int32, (1, B), 1)
    onehot = (bidx == gids).astype(jnp.float32)
    x_rep = jnp.dot(onehot, g_emb_ref[...],
                    preferred_element_type=jnp.float32)
    x_out_ref[...] = jnp.concatenate([x_rep, cand], axis=1)
    h = jnp.dot(x_rep, w0a_ref[...], preferred_element_type=jnp.float32)
    h = h + jnp.dot(cand, w0b_ref[...], preferred_element_type=jnp.float32)
    h = jnp.maximum(h + b0_ref[...], 0.0)
    h = jnp.maximum(jnp.dot(h, w1_ref[...], preferred_element_type=jnp.float32)
                    + b1_ref[...], 0.0)
    h = jnp.maximum(jnp.dot(h, w2_ref[...], preferred_element_type=jnp.float32)
                    + b2_ref[...], 0.0)
    logits = jnp.dot(h, wf_ref[...], preferred_element_type=jnp.float32) \
        + bf_ref[...]
    logits_ref[...] = logits

    masked = jnp.where(onehot > 0.0, logits, _NEG_INF)
    blk_max = jnp.max(masked, axis=0, keepdims=True)
    blk_cnt = jnp.sum(onehot, axis=0, keepdims=True)

    @pl.when(step == 0)
    def _init():
        gmax_ref[...] = jnp.full((1, B), _NEG_INF, jnp.float32)
        counts_ref[...] = jnp.zeros((1, B), jnp.float32)

    gmax_ref[...] = jnp.maximum(gmax_ref[...], blk_max)
    counts_ref[...] = counts_ref[...] + blk_cnt


def _pass_b(bidx_ref, logits_ref, gmax_t_ref, gsum_ref):
    step = pl.program_id(0)
    bidx = bidx_ref[...]
    gids = jax.lax.broadcasted_iota(jnp.int32, (1, B), 1)
    onehot = (bidx == gids).astype(jnp.float32)
    gmax_elem = jnp.dot(onehot, gmax_t_ref[...],
                        preferred_element_type=jnp.float32)
    ex = jnp.exp(logits_ref[...] - gmax_elem)
    ones_row = jnp.ones((1, BLK), jnp.float32)
    blk_sum = jnp.dot(ones_row, onehot * ex,
                      preferred_element_type=jnp.float32)

    @pl.when(step == 0)
    def _init():
        gsum_ref[...] = jnp.zeros((1, B), jnp.float32)

    gsum_ref[...] = gsum_ref[...] + blk_sum


def _pass_c(bidx_ref, bidx2d_ref, logits_ref, gmax_t_ref, gsum_t_ref,
            counts_ref, probs_ref, lp_out_ref, act_out_ref, shift_out_ref,
            bval_ref, bidx_best_ref, bprob_ref, p0_ref):
    step = pl.program_id(0)
    R2 = BLK // 128

    bidx2d = bidx2d_ref[...]
    r2d = jax.lax.broadcasted_iota(jnp.int32, (R2, 128), 0)
    c2d = jax.lax.broadcasted_iota(jnp.int32, (R2, 128), 1)
    j2d = step * BLK + r2d * 128 + c2d
    m2d = ((B - 1) - bidx2d) * N + j2d
    gum2d = _gumbel_from_bits(_threefry_bits(m2d.astype(jnp.uint32)))

    jrow = jax.lax.broadcasted_iota(jnp.int32, (BLK, R2), 0)
    rcol = jax.lax.broadcasted_iota(jnp.int32, (BLK, R2), 1)
    sel_row = ((jrow // 128) == rcol).astype(jnp.float32)
    colmat = jnp.dot(sel_row, gum2d,
                     preferred_element_type=jnp.float32)
    ji = jax.lax.broadcasted_iota(jnp.int32, (BLK, 128), 0)
    ci = jax.lax.broadcasted_iota(jnp.int32, (BLK, 128), 1)
    lane_pick = ((ji % 128) == ci).astype(jnp.float32)
    ones_col = jnp.ones((128, 1), jnp.float32)
    gumbel = jnp.dot(colmat * lane_pick, ones_col,
                     preferred_element_type=jnp.float32)

    bidx = bidx_ref[...]
    gids = jax.lax.broadcasted_iota(jnp.int32, (1, B), 1)
    onehot_b = (bidx == gids)
    onehot = onehot_b.astype(jnp.float32)
    gmax_elem = jnp.dot(onehot, gmax_t_ref[...],
                        preferred_element_type=jnp.float32)
    gsum_elem = jnp.dot(onehot, gsum_t_ref[...],
                        preferred_element_type=jnp.float32) + EPS
    logits = logits_ref[...]
    probs = jnp.exp(logits - gmax_elem) / gsum_elem
    probs_ref[...] = probs

    ok = probs > EPS
    val = jnp.where(ok, logits + gumbel, _NEG_INF)
    vmask = jnp.where(onehot_b, val, _NEG_INF)
    blk_max = jnp.max(vmask, axis=0, keepdims=True)
    is_max = ((vmask == blk_max) & (blk_max > _NEG_INF)).astype(jnp.float32)
    j_col = (jax.lax.broadcasted_iota(jnp.int32, (BLK, 1), 0)
             + step * BLK).astype(jnp.float32)
    ones_row = jnp.ones((1, BLK), jnp.float32)
    blk_arg = jnp.dot(ones_row, is_max * j_col,
                      preferred_element_type=jnp.float32)
    blk_prob = jnp.dot(ones_row, is_max * probs,
                       preferred_element_type=jnp.float32)

    @pl.when(step == 0)
    def _init():
        bval_ref[...] = jnp.full((1, B), _NEG_INF, jnp.float32)
        bidx_best_ref[...] = jnp.zeros((1, B), jnp.float32)
        bprob_ref[...] = jnp.zeros((1, B), jnp.float32)
        p0_ref[...] = probs[0:1, :]

    upd = blk_max > bval_ref[...]
    bval_ref[...] = jnp.where(upd, blk_max, bval_ref[...])
    bidx_best_ref[...] = jnp.where(upd, blk_arg, bidx_best_ref[...])
    bprob_ref[...] = jnp.where(upd, blk_prob, bprob_ref[...])

    @pl.when(step == NBLK - 1)
    def _fin():
        never = bval_ref[...] == _NEG_INF
        sel_prob = jnp.where(never, p0_ref[...], bprob_ref[...])
        sel_idx = jnp.where(never, 0.0, bidx_best_ref[...])
        rev = (jax.lax.broadcasted_iota(jnp.int32, (B, B), 0)
               + jax.lax.broadcasted_iota(jnp.int32, (B, B), 1)) == (B - 1)
        revf = rev.astype(jnp.float32)
        shifted_f = jnp.dot(sel_idx, revf,
                            preferred_element_type=jnp.float32)
        lp = jnp.log(jnp.dot(sel_prob, revf,
                             preferred_element_type=jnp.float32))
        tri = (jax.lax.broadcasted_iota(jnp.int32, (B, B), 0)
               < jax.lax.broadcasted_iota(jnp.int32, (B, B), 1))
        shift_f = jnp.dot(counts_ref[...], tri.astype(jnp.float32),
                          preferred_element_type=jnp.float32)
        shift_out_ref[...] = shifted_f.astype(jnp.int32)
        act_out_ref[...] = (shifted_f - shift_f).astype(jnp.int32)
        lp_out_ref[...] = lp


def kernel(g_emb, g_candidates_emb, batch_idx, W0, b0, W1, b1, W2, b2, Wf, bf):
    bidx2 = batch_idx.reshape(N, 1)
    w0a = W0[:EMB]
    w0b = W0[EMB:]
    b0r = b0.reshape(1, HID)
    b1r = b1.reshape(1, HID)
    b2r = b2.reshape(1, HID)
    bfr = bf.reshape(1, 1)

    row_spec = pl.BlockSpec((BLK, 1), lambda i: (i, 0))
    full = lambda shape: pl.BlockSpec(shape, lambda i: tuple(0 for _ in shape))
    acc = pl.BlockSpec((1, B), lambda i: (0, 0))

    x_states, logits, gmax, counts = pl.pallas_call(
        _pass_a,
        grid=(NBLK,),
        in_specs=[
            full((B, EMB)), row_spec,
            pl.BlockSpec((BLK, EMB), lambda i: (i, 0)),
            full((EMB, HID)), full((EMB, HID)), full((1, HID)),
            full((HID, HID)), full((1, HID)),
            full((HID, HID)), full((1, HID)),
            full((HID, 1)), full((1, 1)),
        ],
        out_specs=[
            pl.BlockSpec((BLK, 2 * EMB), lambda i: (i, 0)),
            row_spec, acc, acc,
        ],
        out_shape=[
            jax.ShapeDtypeStruct((N, 2 * EMB), jnp.float32),
            jax.ShapeDtypeStruct((N, 1), jnp.float32),
            jax.ShapeDtypeStruct((1, B), jnp.float32),
            jax.ShapeDtypeStruct((1, B), jnp.float32),
        ],
    )(g_emb, bidx2, g_candidates_emb, w0a, w0b, b0r, W1, b1r, W2, b2r, Wf, bfr)

    gmax_t = gmax.reshape(B, 1)
    (gsum,) = pl.pallas_call(
        _pass_b,
        grid=(NBLK,),
        in_specs=[row_spec, row_spec, full((B, 1))],
        out_specs=[acc],
        out_shape=[jax.ShapeDtypeStruct((1, B), jnp.float32)],
    )(bidx2, logits, gmax_t)

    bidx2d = batch_idx.reshape(N // 128, 128)
    gsum_t = gsum.reshape(B, 1)
    probs, logprobs, actions, shifted = pl.pallas_call(
        _pass_c,
        grid=(NBLK,),
        in_specs=[
            row_spec,
            pl.BlockSpec((BLK // 128, 128), lambda i: (i, 0)),
            row_spec, full((B, 1)), full((B, 1)), acc,
        ],
        out_specs=[row_spec, acc, acc, acc],
        out_shape=[
            jax.ShapeDtypeStruct((N, 1), jnp.float32),
            jax.ShapeDtypeStruct((1, B), jnp.float32),
            jax.ShapeDtypeStruct((1, B), jnp.int32),
            jax.ShapeDtypeStruct((1, B), jnp.int32),
        ],
        scratch_shapes=[
            pltpu.VMEM((1, B), jnp.float32),
            pltpu.VMEM((1, B), jnp.float32),
            pltpu.VMEM((1, B), jnp.float32),
            pltpu.VMEM((1, 1), jnp.float32),
        ],
    )(bidx2, bidx2d, logits, gmax_t, gsum_t, counts)

    return (g_emb, x_states, probs.reshape(N),
            logprobs.reshape(B), actions.reshape(B), shifted.reshape(B))


# speedup vs baseline: 8.0494x; 1.1165x over previous
import functools

import jax
import jax.numpy as jnp
import numpy as np
from jax.experimental import pallas as pl
from jax.experimental.pallas import tpu as pltpu

B = 128
N = 65536
EMB = 64
HID = 128
EPS = 1e-4

BLK = 4096
NBLK = N // BLK

_NEG_INF = np.float32(-np.inf)
_TINY = np.float32(np.finfo(np.float32).tiny)


def _u32(x):
    return jnp.uint32(x)


def _threefry_bits(m):
    k1 = _u32(0)
    k2 = _u32(123)
    ks2 = k1 ^ k2 ^ _u32(0x1BD11BDA)

    def rotl(x, r):
        return (x << _u32(r)) | (x >> _u32(32 - r))

    x0 = jnp.zeros_like(m) + k1
    x1 = m + k2
    rots = ((13, 15, 26, 6), (17, 29, 16, 24))
    ks = (k1, k2, ks2)
    for i in range(5):
        for r in rots[i % 2]:
            x0 = x0 + x1
            x1 = rotl(x1, r)
            x1 = x1 ^ x0
        x0 = x0 + ks[(i + 1) % 3]
        x1 = x1 + ks[(i + 2) % 3] + _u32(i + 1)
    return x0 ^ x1


def _gumbel_from_bits(bits):
    fb = (bits >> _u32(9)) | _u32(0x3F800000)
    u01 = jax.lax.bitcast_convert_type(fb, jnp.float32) - jnp.float32(1.0)
    one_minus_tiny = jnp.float32(1.0) - _TINY
    u = jnp.maximum(_TINY, u01 * one_minus_tiny + _TINY)
    return -jnp.log(-jnp.log(u))


def _pass_a(g_emb_ref, bidx_ref, cand_ref, w0a_ref, w0b_ref, b0_ref,
            w1_ref, b1_ref, w2_ref, b2_ref, wf_ref, bf_ref,
            x_out_ref, logits_ref, gmax_ref, counts_ref):
    step = pl.program_id(0)
    bidx = bidx_ref[...]
    cand = cand_ref[...]
    gids = jax.lax.broadcasted_iota(jnp.int32, (1, B), 1)
    onehot = (bidx == gids).astype(jnp.float32)
    x_rep = jnp.dot(onehot, g_emb_ref[...],
                    preferred_element_type=jnp.float32)
    x_out_ref[...] = jnp.concatenate([x_rep, cand], axis=1)
    h = jnp.dot(x_rep, w0a_ref[...], preferred_element_type=jnp.float32)
    h = h + jnp.dot(cand, w0b_ref[...], preferred_element_type=jnp.float32)
    h = jnp.maximum(h + b0_ref[...], 0.0)
    h = jnp.maximum(jnp.dot(h, w1_ref[...], preferred_element_type=jnp.float32)
                    + b1_ref[...], 0.0)
    h = jnp.maximum(jnp.dot(h, w2_ref[...], preferred_element_type=jnp.float32)
                    + b2_ref[...], 0.0)
    logits = jnp.dot(h, wf_ref[...], preferred_element_type=jnp.float32) \
        + bf_ref[...]
    logits_ref[...] = logits

    masked = jnp.where(onehot > 0.0, logits, _NEG_INF)
    blk_max = jnp.max(masked, axis=0, keepdims=True)
    blk_cnt = jnp.sum(onehot, axis=0, keepdims=True)

    @pl.when(step == 0)
    def _init():
        gmax_ref[...] = jnp.full((1, B), _NEG_INF, jnp.float32)
        counts_ref[...] = jnp.zeros((1, B), jnp.float32)

    gmax_ref[...] = jnp.maximum(gmax_ref[...], blk_max)
    counts_ref[...] = counts_ref[...] + blk_cnt


def _pass_b(bidx_ref, logits_ref, gmax_t_ref, gsum_ref):
    step = pl.program_id(0)
    bidx = bidx_ref[...]
    gids = jax.lax.broadcasted_iota(jnp.int32, (1, B), 1)
    onehot = (bidx == gids).astype(jnp.float32)
    gmax_elem = jnp.dot(onehot, gmax_t_ref[...],
                        preferred_element_type=jnp.float32)
    ex = jnp.exp(logits_ref[...] - gmax_elem)
    ones_row = jnp.ones((1, BLK), jnp.float32)
    blk_sum = jnp.dot(ones_row, onehot * ex,
                      preferred_element_type=jnp.float32)

    @pl.when(step == 0)
    def _init():
        gsum_ref[...] = jnp.zeros((1, B), jnp.float32)

    gsum_ref[...] = gsum_ref[...] + blk_sum


def _pass_c(bidx_ref, bidx2d_ref, logits_ref, gmax_t_ref, gsum_t_ref,
            counts_ref, probs_ref, lp_out_ref, act_out_ref, shift_out_ref,
            bval_ref, bidx_best_ref, bprob_ref, p0_ref):
    step = pl.program_id(0)
    R2 = BLK // 128

    bidx2d = bidx2d_ref[...]
    r2d = jax.lax.broadcasted_iota(jnp.int32, (R2, 128), 0)
    c2d = jax.lax.broadcasted_iota(jnp.int32, (R2, 128), 1)
    j2d = step * BLK + r2d * 128 + c2d
    m2d = ((B - 1) - bidx2d) * N + j2d
    gum2d = _gumbel_from_bits(_threefry_bits(m2d.astype(jnp.uint32)))

    jrow = jax.lax.broadcasted_iota(jnp.int32, (BLK, R2), 0)
    rcol = jax.lax.broadcasted_iota(jnp.int32, (BLK, R2), 1)
    sel_row = ((jrow // 128) == rcol).astype(jnp.float32)
    colmat = jnp.dot(sel_row, gum2d,
                     preferred_element_type=jnp.float32)
    ji = jax.lax.broadcasted_iota(jnp.int32, (BLK, 128), 0)
    ci = jax.lax.broadcasted_iota(jnp.int32, (BLK, 128), 1)
    lane_pick = ((ji % 128) == ci).astype(jnp.float32)
    ones_col = jnp.ones((128, 1), jnp.float32)
    gumbel = jnp.dot(colmat * lane_pick, ones_col,
                     preferred_element_type=jnp.float32)

    bidx = bidx_ref[...]
    gids = jax.lax.broadcasted_iota(jnp.int32, (1, B), 1)
    onehot_b = (bidx == gids)
    onehot = onehot_b.astype(jnp.float32)
    gmax_elem = jnp.dot(onehot, gmax_t_ref[...],
                        preferred_element_type=jnp.float32)
    gsum_elem = jnp.dot(onehot, gsum_t_ref[...],
                        preferred_element_type=jnp.float32) + EPS
    logits = logits_ref[...]
    probs = jnp.exp(logits - gmax_elem) / gsum_elem
    probs_ref[...] = probs

    ok = probs > EPS
    val = jnp.where(ok, logits + gumbel, _NEG_INF)
    vmask = jnp.where(onehot_b, val, _NEG_INF)
    blk_max = jnp.max(vmask, axis=0, keepdims=True)
    is_max = ((vmask == blk_max) & (blk_max > _NEG_INF)).astype(jnp.float32)
    j_col = (jax.lax.broadcasted_iota(jnp.int32, (BLK, 1), 0)
             + step * BLK).astype(jnp.float32)
    ones_row = jnp.ones((1, BLK), jnp.float32)
    blk_arg = jnp.dot(ones_row, is_max * j_col,
                      preferred_element_type=jnp.float32)
    blk_prob = jnp.dot(ones_row, is_max * probs,
                       preferred_element_type=jnp.float32)

    @pl.when(step == 0)
    def _init():
        bval_ref[...] = jnp.full((1, B), _NEG_INF, jnp.float32)
        bidx_best_ref[...] = jnp.zeros((1, B), jnp.float32)
        bprob_ref[...] = jnp.zeros((1, B), jnp.float32)
        p0_ref[...] = probs[0:1, :]

    upd = blk_max > bval_ref[...]
    bval_ref[...] = jnp.where(upd, blk_max, bval_ref[...])
    bidx_best_ref[...] = jnp.where(upd, blk_arg, bidx_best_ref[...])
    bprob_ref[...] = jnp.where(upd, blk_prob, bprob_ref[...])

    @pl.when(step == NBLK - 1)
    def _fin():
        never = bval_ref[...] == _NEG_INF
        sel_prob = jnp.where(never, p0_ref[...], bprob_ref[...])
        sel_idx = jnp.where(never, 0.0, bidx_best_ref[...])
        rev = (jax.lax.broadcasted_iota(jnp.int32, (B, B), 0)
               + jax.lax.broadcasted_iota(jnp.int32, (B, B), 1)) == (B - 1)
        revf = rev.astype(jnp.float32)
        shifted_f = jnp.dot(sel_idx, revf,
                            preferred_element_type=jnp.float32)
        lp = jnp.log(jnp.dot(sel_prob, revf,
                             preferred_element_type=jnp.float32))
        tri = (jax.lax.broadcasted_iota(jnp.int32, (B, B), 0)
               < jax.lax.broadcasted_iota(jnp.int32, (B, B), 1))
        shift_f = jnp.dot(counts_ref[...], tri.astype(jnp.float32),
                          preferred_element_type=jnp.float32)
        shift_out_ref[...] = shifted_f.astype(jnp.int32)
        act_out_ref[...] = (shifted_f - shift_f).astype(jnp.int32)
        lp_out_ref[...] = lp


def kernel(g_emb, g_candidates_emb, batch_idx, W0, b0, W1, b1, W2, b2, Wf, bf):
    bidx2 = batch_idx.reshape(N, 1)
    w0a = W0[:EMB]
    w0b = W0[EMB:]
    b0r = b0.reshape(1, HID)
    b1r = b1.reshape(1, HID)
    b2r = b2.reshape(1, HID)
    bfr = bf.reshape(1, 1)

    row_spec = pl.BlockSpec((BLK, 1), lambda i: (i, 0))
    full = lambda shape: pl.BlockSpec(shape, lambda i: tuple(0 for _ in shape))
    acc = pl.BlockSpec((1, B), lambda i: (0, 0))

    x_states, logits, gmax, counts = pl.pallas_call(
        _pass_a,
        grid=(NBLK,),
        in_specs=[
            full((B, EMB)), row_spec,
            pl.BlockSpec((BLK, EMB), lambda i: (i, 0)),
            full((EMB, HID)), full((EMB, HID)), full((1, HID)),
            full((HID, HID)), full((1, HID)),
            full((HID, HID)), full((1, HID)),
            full((HID, 1)), full((1, 1)),
        ],
        out_specs=[
            pl.BlockSpec((BLK, 2 * EMB), lambda i: (i, 0)),
            row_spec, acc, acc,
        ],
        out_shape=[
            jax.ShapeDtypeStruct((N, 2 * EMB), jnp.float32),
            jax.ShapeDtypeStruct((N, 1), jnp.float32),
            jax.ShapeDtypeStruct((1, B), jnp.float32),
            jax.ShapeDtypeStruct((1, B), jnp.float32),
        ],
    )(g_emb, bidx2, g_candidates_emb, w0a, w0b, b0r, W1, b1r, W2, b2r, Wf, bfr)

    gmax_t = gmax.reshape(B, 1)
    (gsum,) = pl.pallas_call(
        _pass_b,
        grid=(NBLK,),
        in_specs=[row_spec, row_spec, full((B, 1))],
        out_specs=[acc],
        out_shape=[jax.ShapeDtypeStruct((1, B), jnp.float32)],
    )(bidx2, logits, gmax_t)

    bidx2d = batch_idx.reshape(N // 128, 128)
    gsum_t = gsum.reshape(B, 1)
    probs, logprobs, actions, shifted = pl.pallas_call(
        _pass_c,
        grid=(NBLK,),
        in_specs=[
            row_spec,
            pl.BlockSpec((BLK // 128, 128), lambda i: (i, 0)),
            row_spec, full((B, 1)), full((B, 1)), acc,
        ],
        out_specs=[row_spec, acc, acc, acc],
        out_shape=[
            jax.ShapeDtypeStruct((N, 1), jnp.float32),
            jax.ShapeDtypeStruct((1, B), jnp.float32),
            jax.ShapeDtypeStruct((1, B), jnp.int32),
            jax.ShapeDtypeStruct((1, B), jnp.int32),
        ],
        scratch_shapes=[
            pltpu.VMEM((1, B), jnp.float32),
            pltpu.VMEM((1, B), jnp.float32),
            pltpu.VMEM((1, B), jnp.float32),
            pltpu.VMEM((1, 1), jnp.float32),
        ],
    )(bidx2, bidx2d, logits, gmax_t, gsum_t, counts)

    return (g_emb, x_states, probs.reshape(N),
            logprobs.reshape(B), actions.reshape(B), shifted.reshape(B))


# speedup vs baseline: 8.3161x; 1.0331x over previous
import functools

import jax
import jax.numpy as jnp
import numpy as np
from jax.experimental import pallas as pl
from jax.experimental.pallas import tpu as pltpu

B = 128
N = 65536
EMB = 64
HID = 128
EPS = 1e-4

BLK = 8192
NBLK = N // BLK

_NEG_INF = np.float32(-np.inf)
_TINY = np.float32(np.finfo(np.float32).tiny)


def _u32(x):
    return jnp.uint32(x)


def _threefry_bits(m):
    k1 = _u32(0)
    k2 = _u32(123)
    ks2 = k1 ^ k2 ^ _u32(0x1BD11BDA)

    def rotl(x, r):
        return (x << _u32(r)) | (x >> _u32(32 - r))

    x0 = jnp.zeros_like(m) + k1
    x1 = m + k2
    rots = ((13, 15, 26, 6), (17, 29, 16, 24))
    ks = (k1, k2, ks2)
    for i in range(5):
        for r in rots[i % 2]:
            x0 = x0 + x1
            x1 = rotl(x1, r)
            x1 = x1 ^ x0
        x0 = x0 + ks[(i + 1) % 3]
        x1 = x1 + ks[(i + 2) % 3] + _u32(i + 1)
    return x0 ^ x1


def _gumbel_from_bits(bits):
    fb = (bits >> _u32(9)) | _u32(0x3F800000)
    u01 = jax.lax.bitcast_convert_type(fb, jnp.float32) - jnp.float32(1.0)
    one_minus_tiny = jnp.float32(1.0) - _TINY
    u = jnp.maximum(_TINY, u01 * one_minus_tiny + _TINY)
    return -jnp.log(-jnp.log(u))


def _pass_a(g_emb_ref, bidx_ref, cand_ref, w0a_ref, w0b_ref, b0_ref,
            w1_ref, b1_ref, w2_ref, b2_ref, wf_ref, bf_ref,
            x_out_ref, logits_ref, gmax_ref, counts_ref):
    step = pl.program_id(0)
    bidx = bidx_ref[...]
    cand = cand_ref[...]
    gids = jax.lax.broadcasted_iota(jnp.int32, (1, B), 1)
    onehot = (bidx == gids).astype(jnp.float32)
    x_rep = jnp.dot(onehot, g_emb_ref[...],
                    preferred_element_type=jnp.float32)
    x_out_ref[...] = jnp.concatenate([x_rep, cand], axis=1)
    h = jnp.dot(x_rep, w0a_ref[...], preferred_element_type=jnp.float32)
    h = h + jnp.dot(cand, w0b_ref[...], preferred_element_type=jnp.float32)
    h = jnp.maximum(h + b0_ref[...], 0.0)
    h = jnp.maximum(jnp.dot(h, w1_ref[...], preferred_element_type=jnp.float32)
                    + b1_ref[...], 0.0)
    h = jnp.maximum(jnp.dot(h, w2_ref[...], preferred_element_type=jnp.float32)
                    + b2_ref[...], 0.0)
    logits = jnp.dot(h, wf_ref[...], preferred_element_type=jnp.float32) \
        + bf_ref[...]
    logits_ref[...] = logits

    masked = jnp.where(onehot > 0.0, logits, _NEG_INF)
    blk_max = jnp.max(masked, axis=0, keepdims=True)
    blk_cnt = jnp.sum(onehot, axis=0, keepdims=True)

    @pl.when(step == 0)
    def _init():
        gmax_ref[...] = jnp.full((1, B), _NEG_INF, jnp.float32)
        counts_ref[...] = jnp.zeros((1, B), jnp.float32)

    gmax_ref[...] = jnp.maximum(gmax_ref[...], blk_max)
    counts_ref[...] = counts_ref[...] + blk_cnt


def _pass_b(bidx_ref, logits_ref, gmax_t_ref, gsum_ref):
    step = pl.program_id(0)
    bidx = bidx_ref[...]
    gids = jax.lax.broadcasted_iota(jnp.int32, (1, B), 1)
    onehot = (bidx == gids).astype(jnp.float32)
    gmax_elem = jnp.dot(onehot, gmax_t_ref[...],
                        preferred_element_type=jnp.float32)
    ex = jnp.exp(logits_ref[...] - gmax_elem)
    ones_row = jnp.ones((1, BLK), jnp.float32)
    blk_sum = jnp.dot(ones_row, onehot * ex,
                      preferred_element_type=jnp.float32)

    @pl.when(step == 0)
    def _init():
        gsum_ref[...] = jnp.zeros((1, B), jnp.float32)

    gsum_ref[...] = gsum_ref[...] + blk_sum


def _pass_c(bidx_ref, bidx2d_ref, logits_ref, gmax_t_ref, gsum_t_ref,
            counts_ref, probs_ref, lp_out_ref, act_out_ref, shift_out_ref,
            bval_ref, bidx_best_ref, bprob_ref, p0_ref):
    step = pl.program_id(0)
    R2 = BLK // 128

    bidx2d = bidx2d_ref[...]
    r2d = jax.lax.broadcasted_iota(jnp.int32, (R2, 128), 0)
    c2d = jax.lax.broadcasted_iota(jnp.int32, (R2, 128), 1)
    j2d = step * BLK + r2d * 128 + c2d
    m2d = ((B - 1) - bidx2d) * N + j2d
    gum2d = _gumbel_from_bits(_threefry_bits(m2d.astype(jnp.uint32)))

    jrow = jax.lax.broadcasted_iota(jnp.int32, (BLK, R2), 0)
    rcol = jax.lax.broadcasted_iota(jnp.int32, (BLK, R2), 1)
    sel_row = ((jrow // 128) == rcol).astype(jnp.float32)
    colmat = jnp.dot(sel_row, gum2d,
                     preferred_element_type=jnp.float32)
    ji = jax.lax.broadcasted_iota(jnp.int32, (BLK, 128), 0)
    ci = jax.lax.broadcasted_iota(jnp.int32, (BLK, 128), 1)
    lane_pick = ((ji % 128) == ci).astype(jnp.float32)
    ones_col = jnp.ones((128, 1), jnp.float32)
    gumbel = jnp.dot(colmat * lane_pick, ones_col,
                     preferred_element_type=jnp.float32)

    bidx = bidx_ref[...]
    gids = jax.lax.broadcasted_iota(jnp.int32, (1, B), 1)
    onehot_b = (bidx == gids)
    onehot = onehot_b.astype(jnp.float32)
    gmax_elem = jnp.dot(onehot, gmax_t_ref[...],
                        preferred_element_type=jnp.float32)
    gsum_elem = jnp.dot(onehot, gsum_t_ref[...],
                        preferred_element_type=jnp.float32) + EPS
    logits = logits_ref[...]
    probs = jnp.exp(logits - gmax_elem) / gsum_elem
    probs_ref[...] = probs

    ok = probs > EPS
    val = jnp.where(ok, logits + gumbel, _NEG_INF)
    vmask = jnp.where(onehot_b, val, _NEG_INF)
    blk_max = jnp.max(vmask, axis=0, keepdims=True)
    is_max = ((vmask == blk_max) & (blk_max > _NEG_INF)).astype(jnp.float32)
    j_col = (jax.lax.broadcasted_iota(jnp.int32, (BLK, 1), 0)
             + step * BLK).astype(jnp.float32)
    ones_row = jnp.ones((1, BLK), jnp.float32)
    blk_arg = jnp.dot(ones_row, is_max * j_col,
                      preferred_element_type=jnp.float32)
    blk_prob = jnp.dot(ones_row, is_max * probs,
                       preferred_element_type=jnp.float32)

    @pl.when(step == 0)
    def _init():
        bval_ref[...] = jnp.full((1, B), _NEG_INF, jnp.float32)
        bidx_best_ref[...] = jnp.zeros((1, B), jnp.float32)
        bprob_ref[...] = jnp.zeros((1, B), jnp.float32)
        p0_ref[...] = probs[0:1, :]

    upd = blk_max > bval_ref[...]
    bval_ref[...] = jnp.where(upd, blk_max, bval_ref[...])
    bidx_best_ref[...] = jnp.where(upd, blk_arg, bidx_best_ref[...])
    bprob_ref[...] = jnp.where(upd, blk_prob, bprob_ref[...])

    @pl.when(step == NBLK - 1)
    def _fin():
        never = bval_ref[...] == _NEG_INF
        sel_prob = jnp.where(never, p0_ref[...], bprob_ref[...])
        sel_idx = jnp.where(never, 0.0, bidx_best_ref[...])
        rev = (jax.lax.broadcasted_iota(jnp.int32, (B, B), 0)
               + jax.lax.broadcasted_iota(jnp.int32, (B, B), 1)) == (B - 1)
        revf = rev.astype(jnp.float32)
        shifted_f = jnp.dot(sel_idx, revf,
                            preferred_element_type=jnp.float32)
        lp = jnp.log(jnp.dot(sel_prob, revf,
                             preferred_element_type=jnp.float32))
        tri = (jax.lax.broadcasted_iota(jnp.int32, (B, B), 0)
               < jax.lax.broadcasted_iota(jnp.int32, (B, B), 1))
        shift_f = jnp.dot(counts_ref[...], tri.astype(jnp.float32),
                          preferred_element_type=jnp.float32)
        shift_out_ref[...] = shifted_f.astype(jnp.int32)
        act_out_ref[...] = (shifted_f - shift_f).astype(jnp.int32)
        lp_out_ref[...] = lp


def kernel(g_emb, g_candidates_emb, batch_idx, W0, b0, W1, b1, W2, b2, Wf, bf):
    bidx2 = batch_idx.reshape(N, 1)
    w0a = W0[:EMB]
    w0b = W0[EMB:]
    b0r = b0.reshape(1, HID)
    b1r = b1.reshape(1, HID)
    b2r = b2.reshape(1, HID)
    bfr = bf.reshape(1, 1)

    row_spec = pl.BlockSpec((BLK, 1), lambda i: (i, 0))
    full = lambda shape: pl.BlockSpec(shape, lambda i: tuple(0 for _ in shape))
    acc = pl.BlockSpec((1, B), lambda i: (0, 0))

    x_states, logits, gmax, counts = pl.pallas_call(
        _pass_a,
        grid=(NBLK,),
        in_specs=[
            full((B, EMB)), row_spec,
            pl.BlockSpec((BLK, EMB), lambda i: (i, 0)),
            full((EMB, HID)), full((EMB, HID)), full((1, HID)),
            full((HID, HID)), full((1, HID)),
            full((HID, HID)), full((1, HID)),
            full((HID, 1)), full((1, 1)),
        ],
        out_specs=[
            pl.BlockSpec((BLK, 2 * EMB), lambda i: (i, 0)),
            row_spec, acc, acc,
        ],
        out_shape=[
            jax.ShapeDtypeStruct((N, 2 * EMB), jnp.float32),
            jax.ShapeDtypeStruct((N, 1), jnp.float32),
            jax.ShapeDtypeStruct((1, B), jnp.float32),
            jax.ShapeDtypeStruct((1, B), jnp.float32),
        ],
    )(g_emb, bidx2, g_candidates_emb, w0a, w0b, b0r, W1, b1r, W2, b2r, Wf, bfr)

    gmax_t = gmax.reshape(B, 1)
    (gsum,) = pl.pallas_call(
        _pass_b,
        grid=(NBLK,),
        in_specs=[row_spec, row_spec, full((B, 1))],
        out_specs=[acc],
        out_shape=[jax.ShapeDtypeStruct((1, B), jnp.float32)],
    )(bidx2, logits, gmax_t)

    bidx2d = batch_idx.reshape(N // 128, 128)
    gsum_t = gsum.reshape(B, 1)
    probs, logprobs, actions, shifted = pl.pallas_call(
        _pass_c,
        grid=(NBLK,),
        in_specs=[
            row_spec,
            pl.BlockSpec((BLK // 128, 128), lambda i: (i, 0)),
            row_spec, full((B, 1)), full((B, 1)), acc,
        ],
        out_specs=[row_spec, acc, acc, acc],
        out_shape=[
            jax.ShapeDtypeStruct((N, 1), jnp.float32),
            jax.ShapeDtypeStruct((1, B), jnp.float32),
            jax.ShapeDtypeStruct((1, B), jnp.int32),
            jax.ShapeDtypeStruct((1, B), jnp.int32),
        ],
        scratch_shapes=[
            pltpu.VMEM((1, B), jnp.float32),
            pltpu.VMEM((1, B), jnp.float32),
            pltpu.VMEM((1, B), jnp.float32),
            pltpu.VMEM((1, 1), jnp.float32),
        ],
    )(bidx2, bidx2d, logits, gmax_t, gsum_t, counts)

    return (g_emb, x_states, probs.reshape(N),
            logprobs.reshape(B), actions.reshape(B), shifted.reshape(B))
